# baseline (speedup 1.0000x reference)
import functools

import jax
import jax.numpy as jnp
from jax import lax
from jax.experimental import pallas as pl
from jax.experimental.pallas import tpu as pltpu
from jax.experimental.pallas import tpu_sc as plsc

R = 128
C = 32768
L = 16
NC = 2
NS = 16
NW = NC * NS
RPW = R // NW
UNROLL = 8
NJ = C // (L * UNROLL)
ZWORDS = 4096
NZ = C // ZWORDS

_mesh = plsc.VectorSubcoreMesh(core_axis_name="c", subcore_axis_name="s")


@functools.partial(
    pl.kernel,
    mesh=_mesh,
    out_type=jax.ShapeDtypeStruct((R * C,), jnp.float32),
    scratch_types=[
        pltpu.VMEM((C,), jnp.float32),
        pltpu.VMEM((C,), jnp.float32),
        pltpu.VMEM((ZWORDS,), jnp.float32),
        pltpu.VMEM((RPW * L,), jnp.float32),
        pltpu.SemaphoreType.DMA,
        pltpu.SemaphoreType.DMA,
        pltpu.SemaphoreType.DMA,
        pltpu.SemaphoreType.DMA,
    ],
)
def _keepmax_sc(x_hbm, out_hbm, buf0, buf1, zbuf, pbuf,
                sem_r0, sem_r1, sem_w, sem_p):
    wid = lax.axis_index("s") * NC + lax.axis_index("c")
    row0 = wid * RPW

    iota = lax.iota(jnp.int32, L)
    zero16 = jnp.zeros((L,), jnp.float32)

    def _zfill(i, _):
        zbuf[pl.ds(pl.multiple_of(i * L, L), L)] = zero16
        return 0
    lax.fori_loop(0, ZWORDS // L, _zfill, 0)

    zw = []
    for r in range(RPW):
        rowbase = (row0 + r) * C
        for z in range(NZ):
            dst = out_hbm.at[pl.ds(pl.multiple_of(rowbase + z * ZWORDS, L),
                                   ZWORDS)]
            zw.append(pltpu.async_copy(zbuf, dst, sem_w))

    bufs = (buf0, buf1)
    sems = (sem_r0, sem_r1)
    rd = [None] * RPW
    rd[0] = pltpu.async_copy(
        x_hbm.at[pl.ds(pl.multiple_of(row0 * C, L), C)], buf0, sem_r0)

    segs = []
    for r in range(RPW):
        buf = bufs[r % 2]
        if r + 1 < RPW:
            nxt = (r + 1) % 2
            src = x_hbm.at[pl.ds(pl.multiple_of((row0 + r + 1) * C, L), C)]
            rd[r + 1] = pltpu.async_copy(src, bufs[nxt], sems[nxt])
        rd[r].wait()

        neg_inf = jnp.full((L,), -jnp.inf, jnp.float32)
        bv0 = tuple(neg_inf for _ in range(UNROLL))
        bj0 = tuple(jnp.zeros((L,), jnp.int32) for _ in range(UNROLL))

        def _step(j, carry):
            bvs, bjs = carry
            base = pl.multiple_of(j * (L * UNROLL), L * UNROLL)
            jb = jnp.full((L,), j, jnp.int32)
            nbvs, nbjs = [], []
            for k in range(UNROLL):
                v = buf[pl.ds(base + k * L, L)]
                gt = v > bvs[k]
                nbvs.append(jnp.maximum(bvs[k], v))
                nbjs.append(jnp.where(gt, jb, bjs[k]))
            return tuple(nbvs), tuple(nbjs)

        bvs, bjs = lax.fori_loop(0, NJ, _step, (bv0, bj0))

        mval, mcol = None, None
        for k in range(UNROLL):
            col = bjs[k] * (L * UNROLL) + (k * L + iota)
            if mval is None:
                mval, mcol = bvs[k], col
            else:
                take = (bvs[k] > mval) | ((bvs[k] == mval) & (col < mcol))
                mval = jnp.where(take, bvs[k], mval)
                mcol = jnp.where(take, col, mcol)

        rmax = mval[0]
        rcol = mcol[0]
        for l in range(1, L):
            v = mval[l]
            c = mcol[l]
            take = (v > rmax) | ((v == rmax) & (c < rcol))
            rmax = jnp.where(take, v, rmax)
            rcol = jnp.where(take, c, rcol)

        lane = lax.rem(rcol, L)
        seg = rcol - lane
        pokev = jnp.where(iota == lane, jnp.full((L,), rmax), zero16)
        pbuf[pl.ds(r * L, L)] = pokev
        segs.append(seg)

    for h in zw:
        h.wait()

    pk = []
    for r in range(RPW):
        rowbase = (row0 + r) * C
        dst = out_hbm.at[pl.ds(pl.multiple_of(rowbase + segs[r], L), L)]
        pk.append(pltpu.async_copy(pbuf.at[pl.ds(r * L, L)], dst, sem_p))
    for h in pk:
        h.wait()


def kernel(x):
    out_flat = _keepmax_sc(x.reshape(R * C))
    return out_flat.reshape(R, C)


# speedup vs baseline: 1.9793x; 1.9793x over previous
import functools

import jax
import jax.numpy as jnp
from jax import lax
from jax.experimental import pallas as pl
from jax.experimental.pallas import tpu as pltpu
from jax.experimental.pallas import tpu_sc as plsc

R = 128
C = 32768
L = 16
NC = 2
NS = 16
NW = NC * NS
RPW = R // NW
UNROLL = 8
NJ = C // (L * UNROLL)
ZWORDS = 4096
NZ = C // ZWORDS

_mesh = plsc.VectorSubcoreMesh(core_axis_name="c", subcore_axis_name="s")


@functools.partial(
    pl.kernel,
    mesh=_mesh,
    out_type=jax.ShapeDtypeStruct((R, C), jnp.float32),
    scratch_types=[
        pltpu.VMEM((C,), jnp.float32),
        pltpu.VMEM((C,), jnp.float32),
        pltpu.VMEM((ZWORDS,), jnp.float32),
        pltpu.VMEM((RPW * L,), jnp.float32),
        pltpu.SemaphoreType.DMA,
        pltpu.SemaphoreType.DMA,
        pltpu.SemaphoreType.DMA,
        pltpu.SemaphoreType.DMA,
    ],
)
def _keepmax_sc(x_hbm, out_hbm, buf0, buf1, zbuf, pbuf,
                sem_r0, sem_r1, sem_w, sem_p):
    wid = lax.axis_index("s") * NC + lax.axis_index("c")
    row0 = wid * RPW

    iota = lax.iota(jnp.int32, L)
    zero16 = jnp.zeros((L,), jnp.float32)

    def _zfill(i, _):
        zbuf[pl.ds(pl.multiple_of(i * L, L), L)] = zero16
        return 0
    lax.fori_loop(0, ZWORDS // L, _zfill, 0)

    zw = []
    for r in range(RPW):
        for z in range(NZ):
            dst = out_hbm.at[row0 + r, pl.ds(pl.multiple_of(z * ZWORDS, L),
                                             ZWORDS)]
            zw.append(pltpu.async_copy(zbuf, dst, sem_w))

    bufs = (buf0, buf1)
    sems = (sem_r0, sem_r1)
    rd = [None] * RPW
    rd[0] = pltpu.async_copy(x_hbm.at[row0], buf0, sem_r0)

    segs = []
    for r in range(RPW):
        buf = bufs[r % 2]
        if r + 1 < RPW:
            nxt = (r + 1) % 2
            rd[r + 1] = pltpu.async_copy(x_hbm.at[row0 + r + 1], bufs[nxt],
                                         sems[nxt])
        rd[r].wait()

        neg_inf = jnp.full((L,), -jnp.inf, jnp.float32)
        bv0 = tuple(neg_inf for _ in range(UNROLL))
        bj0 = tuple(jnp.zeros((L,), jnp.int32) for _ in range(UNROLL))

        def _step(j, carry):
            bvs, bjs = carry
            base = pl.multiple_of(j * (L * UNROLL), L * UNROLL)
            jb = jnp.full((L,), j, jnp.int32)
            nbvs, nbjs = [], []
            for k in range(UNROLL):
                v = buf[pl.ds(base + k * L, L)]
                gt = v > bvs[k]
                nbvs.append(jnp.maximum(bvs[k], v))
                nbjs.append(jnp.where(gt, jb, bjs[k]))
            return tuple(nbvs), tuple(nbjs)

        bvs, bjs = lax.fori_loop(0, NJ, _step, (bv0, bj0))

        mval, mcol = None, None
        for k in range(UNROLL):
            col = bjs[k] * (L * UNROLL) + (k * L + iota)
            if mval is None:
                mval, mcol = bvs[k], col
            else:
                take = (bvs[k] > mval) | ((bvs[k] == mval) & (col < mcol))
                mval = jnp.where(take, bvs[k], mval)
                mcol = jnp.where(take, col, mcol)

        rmax = mval[0]
        rcol = mcol[0]
        for l in range(1, L):
            v = mval[l]
            c = mcol[l]
            take = (v > rmax) | ((v == rmax) & (c < rcol))
            rmax = jnp.where(take, v, rmax)
            rcol = jnp.where(take, c, rcol)

        lane = lax.rem(rcol, L)
        seg = rcol - lane
        pokev = jnp.where(iota == lane, jnp.full((L,), rmax), zero16)
        pbuf[pl.ds(r * L, L)] = pokev
        segs.append(seg)

    for h in zw:
        h.wait()

    pk = []
    for r in range(RPW):
        dst = out_hbm.at[row0 + r, pl.ds(pl.multiple_of(segs[r], L), L)]
        pk.append(pltpu.async_copy(pbuf.at[pl.ds(r * L, L)], dst, sem_p))
    for h in pk:
        h.wait()


def kernel(x):
    return _keepmax_sc(x)
